# baseline (speedup 1.0000x reference)
import functools

import jax
import jax.numpy as jnp
from jax import lax
from jax.experimental import pallas as pl
from jax.experimental.pallas import tpu as pltpu
from jax.experimental.pallas import tpu_sc as plsc

DIM = 256
LANES = 16
NSLICE = DIM // LANES
CHUNK = 128


@functools.partial(jax.jit, static_argnames=("n_out",))
def _sc_tokenize(table, idx, vals, params, extra_idx, extra_val, n_out):
    info = plsc.get_sparse_core_info()
    nc, ns = info.num_cores, info.num_subcores
    nw = nc * ns
    rpw = (n_out - 1) // nw
    nch = rpw // CHUNK

    mesh = plsc.VectorSubcoreMesh(core_axis_name="c", subcore_axis_name="s")

    @functools.partial(
        pl.kernel,
        mesh=mesh,
        compiler_params=pltpu.CompilerParams(needs_layout_passes=False),
        out_type=jax.ShapeDtypeStruct((n_out, DIM), jnp.float32),
        scratch_types=[
            pltpu.VMEM((rpw,), jnp.int32),
            pltpu.VMEM((rpw,), jnp.float32),
            pltpu.VMEM((4, DIM), jnp.float32),
            pltpu.VMEM((CHUNK, DIM), jnp.float32),
            pltpu.VMEM((CHUNK, DIM), jnp.float32),
            pltpu.VMEM((8,), jnp.int32),
            pltpu.VMEM((16,), jnp.float32),
            pltpu.VMEM((8, DIM), jnp.float32),
            pltpu.SemaphoreType.DMA,
            pltpu.SemaphoreType.DMA,
        ],
    )
    def k(table_hbm, idx_hbm, vals_hbm, params_hbm, xidx_hbm, xval_hbm,
          out_hbm, idx_v, vals_v, params_v, rows_a, rows_b, xidx_v, xval_v,
          rows_x, sem_a, sem_b):
        wid = lax.axis_index("s") * nc + lax.axis_index("c")
        start = pl.multiple_of(wid * rpw, CHUNK)

        pltpu.sync_copy(params_hbm, params_v)
        pltpu.sync_copy(idx_hbm.at[pl.ds(start, rpw)], idx_v)
        pltpu.sync_copy(vals_hbm.at[pl.ds(start, rpw)], vals_v)

        bufs = (rows_a, rows_b)
        sems = (sem_a, sem_b)
        handles = [None, None]
        handles[0] = pltpu.async_copy(
            table_hbm.at[idx_v.at[pl.ds(0, CHUNK)]], rows_a, sem_a)
        for c in range(nch):
            buf = bufs[c % 2]
            handles[c % 2].wait()
            if c + 1 < nch:
                handles[(c + 1) % 2] = pltpu.async_copy(
                    table_hbm.at[idx_v.at[pl.ds((c + 1) * CHUNK, CHUNK)]],
                    bufs[(c + 1) % 2], sems[(c + 1) % 2])
            base = c * CHUNK
            for d in range(NSLICE):
                sl = pl.ds(d * LANES, LANES)
                w_d = params_v[0, sl]
                b_d = params_v[1, sl]
                wv_d = params_v[2, sl]

                def body(i, carry, buf=buf, w_d=w_d, b_d=b_d, wv_d=wv_d,
                         base=base, sl=sl):
                    s = plsc.load_gather(
                        vals_v, [jnp.full((LANES,), base + i, jnp.int32)])
                    t = jnp.maximum(s * w_d + b_d, 0.0) + wv_d
                    buf[i, sl] = buf[i, sl] + t
                    return carry

                lax.fori_loop(0, CHUNK, body, 0, unroll=4)

            if c == 0:
                @pl.when(wid == 0)
                def _cls_row():
                    for d in range(NSLICE):
                        sl = pl.ds(d * LANES, LANES)
                        buf[0, sl] = params_v[2, sl] + params_v[3, sl]

            pltpu.sync_copy(
                buf, out_hbm.at[pl.ds(start + base, CHUNK)])

        @pl.when(wid == nw - 1)
        def _tail_row():
            pltpu.sync_copy(xidx_hbm, xidx_v)
            pltpu.sync_copy(xval_hbm, xval_v)
            pltpu.async_copy(table_hbm.at[xidx_v], rows_x, sem_a).wait()
            s = xval_v[...]
            for d in range(NSLICE):
                sl = pl.ds(d * LANES, LANES)
                t = jnp.maximum(s * params_v[0, sl] + params_v[1, sl], 0.0)
                rows_x[0, sl] = rows_x[0, sl] + t + params_v[2, sl]
            pltpu.sync_copy(rows_x.at[pl.ds(0, 1)],
                            out_hbm.at[pl.ds(n_out - 1, 1)])

    return k(table, idx, vals, params, extra_idx, extra_val)


def kernel(indices, values, table, wv_omics, cls_token, fc_w, fc_b):
    L = indices.shape[0]
    idx32 = indices.astype(jnp.int32)
    vals32 = values.astype(jnp.float32)
    idx = jnp.concatenate([jnp.zeros((1,), jnp.int32), idx32])
    vals = jnp.concatenate([jnp.zeros((1,), jnp.float32), vals32])
    params = jnp.stack([
        fc_w.reshape(DIM), fc_b.reshape(DIM),
        wv_omics.reshape(DIM), cls_token.reshape(DIM)])
    extra_idx = jnp.full((8,), idx32[-1], jnp.int32)
    extra_val = jnp.full((16,), vals32[-1], jnp.float32)
    out = _sc_tokenize(table, idx, vals, params, extra_idx, extra_val,
                       n_out=L + 1)
    return out[None, :, :]


# speedup vs baseline: 1.5898x; 1.5898x over previous
import functools

import jax
import jax.numpy as jnp
from jax import lax
from jax.experimental import pallas as pl
from jax.experimental.pallas import tpu as pltpu
from jax.experimental.pallas import tpu_sc as plsc

DIM = 256
LANES = 16
NSLICE = DIM // LANES
CHUNK = 128
DG = 4
NBUF = 3
UNROLL = 4


@functools.partial(jax.jit, static_argnames=("n_out",))
def _sc_tokenize(table, idx, vals, params, extra_idx, extra_val, n_out):
    info = plsc.get_sparse_core_info()
    nc, ns = info.num_cores, info.num_subcores
    nw = nc * ns
    rpw = (n_out - 1) // nw
    nch = rpw // CHUNK

    mesh = plsc.VectorSubcoreMesh(core_axis_name="c", subcore_axis_name="s")

    @functools.partial(
        pl.kernel,
        mesh=mesh,
        compiler_params=pltpu.CompilerParams(needs_layout_passes=False),
        out_type=jax.ShapeDtypeStruct((1, n_out, DIM), jnp.float32),
        scratch_types=[
            pltpu.VMEM((rpw,), jnp.int32),
            pltpu.VMEM((rpw,), jnp.float32),
            pltpu.VMEM((4, DIM), jnp.float32),
            [pltpu.VMEM((CHUNK, DIM), jnp.float32)] * NBUF,
            pltpu.VMEM((8,), jnp.int32),
            pltpu.VMEM((16,), jnp.float32),
            pltpu.VMEM((8, DIM), jnp.float32),
            [pltpu.SemaphoreType.DMA] * NBUF,
            [pltpu.SemaphoreType.DMA] * NBUF,
        ],
    )
    def k(table_hbm, idx_hbm, vals_hbm, params_hbm, xidx_hbm, xval_hbm,
          out_hbm, idx_v, vals_v, params_v, bufs, xidx_v, xval_v,
          rows_x, gsems, wsems):
        wid = lax.axis_index("s") * nc + lax.axis_index("c")
        start = pl.multiple_of(wid * rpw, CHUNK)

        pltpu.sync_copy(params_hbm, params_v)
        pltpu.sync_copy(idx_hbm.at[pl.ds(start, rpw)], idx_v)
        pltpu.sync_copy(vals_hbm.at[pl.ds(start, rpw)], vals_v)

        gh = [None] * NBUF
        wh = [None] * NBUF
        gh[0] = pltpu.async_copy(
            table_hbm.at[idx_v.at[pl.ds(0, CHUNK)]], bufs[0], gsems[0])
        for c in range(nch):
            b = c % NBUF
            buf = bufs[b]
            gh[b].wait()
            if c + 1 < nch:
                nb = (c + 1) % NBUF
                if wh[nb] is not None:
                    wh[nb].wait()
                    wh[nb] = None
                gh[nb] = pltpu.async_copy(
                    table_hbm.at[idx_v.at[pl.ds((c + 1) * CHUNK, CHUNK)]],
                    bufs[nb], gsems[nb])
            base = c * CHUNK
            for dg in range(NSLICE // DG):
                sls = [pl.ds((dg * DG + g) * LANES, LANES) for g in range(DG)]
                ws = [params_v[0, sl] for sl in sls]
                bs = [params_v[1, sl] for sl in sls]
                wvs = [params_v[2, sl] for sl in sls]

                def body(i, buf=buf, sls=sls, ws=ws, bs=bs, wvs=wvs,
                         base=base):
                    s = plsc.load_gather(
                        vals_v, [jnp.full((LANES,), base + i, jnp.int32)])
                    for g in range(DG):
                        t = jnp.maximum(s * ws[g] + bs[g], 0.0) + wvs[g]
                        buf[i, sls[g]] = buf[i, sls[g]] + t

                plsc.parallel_loop(0, CHUNK, 1, unroll=UNROLL)(body)

            if c == 0:
                @pl.when(wid == 0)
                def _cls_row():
                    for d in range(NSLICE):
                        sl = pl.ds(d * LANES, LANES)
                        buf[0, sl] = params_v[2, sl] + params_v[3, sl]

            wh[b] = pltpu.async_copy(
                buf, out_hbm.at[0, pl.ds(start + base, CHUNK)], wsems[b])

        @pl.when(wid == nw - 1)
        def _tail_row():
            pltpu.sync_copy(xidx_hbm, xidx_v)
            pltpu.sync_copy(xval_hbm, xval_v)
            pltpu.async_copy(table_hbm.at[xidx_v], rows_x, gsems[0]).wait()
            s = xval_v[...]
            for d in range(NSLICE):
                sl = pl.ds(d * LANES, LANES)
                t = jnp.maximum(s * params_v[0, sl] + params_v[1, sl], 0.0)
                rows_x[0, sl] = rows_x[0, sl] + t + params_v[2, sl]
            pltpu.sync_copy(rows_x.at[pl.ds(0, 1)],
                            out_hbm.at[0, pl.ds(n_out - 1, 1)])

        for b in range(NBUF):
            if wh[b] is not None:
                wh[b].wait()

    return k(table, idx, vals, params, extra_idx, extra_val)


def kernel(indices, values, table, wv_omics, cls_token, fc_w, fc_b):
    L = indices.shape[0]
    idx32 = indices.astype(jnp.int32)
    vals32 = values.astype(jnp.float32)
    idx = jnp.concatenate([jnp.zeros((1,), jnp.int32), idx32])
    vals = jnp.concatenate([jnp.zeros((1,), jnp.float32), vals32])
    params = jnp.stack([
        fc_w.reshape(DIM), fc_b.reshape(DIM),
        wv_omics.reshape(DIM), cls_token.reshape(DIM)])
    extra_idx = jnp.full((8,), idx32[-1], jnp.int32)
    extra_val = jnp.full((16,), vals32[-1], jnp.float32)
    return _sc_tokenize(table, idx, vals, params, extra_idx, extra_val,
                        n_out=L + 1)
